# Initial kernel scaffold; baseline (speedup 1.0000x reference)
import jax
import jax.numpy as jnp
from jax.experimental import pallas as pl


def kernel(node_feat, edge_feat, state_attr, eW1, eb1, eW2, eb2, nW1, nb1, nW2, nb2, sW1, sb1, sW2, sb2, edge_index):
    raise NotImplementedError("write your pallas kernel here")


# speedup vs baseline: 1.1165x; 1.1165x over previous
import functools

import jax
import jax.numpy as jnp
from jax.experimental import pallas as pl
from jax.experimental.pallas import tpu as pltpu

_N = 10000
_E = 320000
_DN = 128
_DE = 16
_DS = 64
_DH = 128

_BN = 2000
_BE = 3200


def _pq_body(nf, wa, wb, p_out, q_out):
    x = nf[...]
    p_out[...] = jnp.dot(x, wa[...], preferred_element_type=jnp.float32)
    q_out[...] = jnp.dot(x, wb[...], preferred_element_type=jnp.float32)


def _edge_body(g, ef, st, wu, we, eb1, ew2, eb2, mij, uesum, acc):
    i = pl.program_id(0)
    ce = jnp.dot(st[...], wu[...], preferred_element_type=jnp.float32) + eb1[...]
    h1 = jnp.maximum(
        g[...] + jnp.dot(ef[...], we[...], preferred_element_type=jnp.float32) + ce,
        0.0)
    m = jnp.maximum(
        jnp.dot(h1, ew2[...], preferred_element_type=jnp.float32) + eb2[...], 0.0)
    mij[...] = m

    @pl.when(i == 0)
    def _():
        acc[...] = jnp.zeros_like(acc)

    acc[...] += jnp.sum(m, axis=0, keepdims=True)

    @pl.when(i == pl.num_programs(0) - 1)
    def _():
        uesum[...] = acc[...]


def _node_body(nf, esum, deg, st, a_nf, a_ve, wun, nb1, nw2, nb2, uesum,
               sA, sB, sC, sb1, sw2, sb2, vnew, snew, accv):
    i = pl.program_id(0)
    ve = esum[...] / jnp.maximum(deg[...], 1.0)
    cn = jnp.dot(st[...], wun[...], preferred_element_type=jnp.float32) + nb1[...]
    h = jnp.maximum(
        jnp.dot(nf[...], a_nf[...], preferred_element_type=jnp.float32)
        + jnp.dot(ve, a_ve[...], preferred_element_type=jnp.float32) + cn, 0.0)
    v = jnp.maximum(
        jnp.dot(h, nw2[...], preferred_element_type=jnp.float32) + nb2[...], 0.0)
    vnew[...] = v

    @pl.when(i == 0)
    def _():
        accv[...] = jnp.zeros_like(accv)

    accv[...] += jnp.sum(v, axis=0, keepdims=True)

    @pl.when(i == pl.num_programs(0) - 1)
    def _():
        u_edge = uesum[...] * (1.0 / _E)
        u_vertex = accv[...] * (1.0 / _N)
        s1 = jnp.maximum(
            jnp.dot(st[...], sA[...], preferred_element_type=jnp.float32)
            + jnp.dot(u_edge, sB[...], preferred_element_type=jnp.float32)
            + jnp.dot(u_vertex, sC[...], preferred_element_type=jnp.float32)
            + sb1[...], 0.0)
        snew[...] = jnp.maximum(
            jnp.dot(s1, sw2[...], preferred_element_type=jnp.float32) + sb2[...],
            0.0)


def _full(shape):
    return pl.BlockSpec(shape, lambda *_: tuple(0 for _ in shape))


def kernel(node_feat, edge_feat, state_attr,
           eW1, eb1, eW2, eb2,
           nW1, nb1, nW2, nb2,
           sW1, sb1, sW2, sb2,
           edge_index):
    src = edge_index[0]
    dst = edge_index[1]
    f32 = jnp.float32

    Wvi = eW1[0:_DN]
    Wvj = eW1[_DN:2 * _DN]
    We = eW1[2 * _DN:2 * _DN + _DE]
    Wu = eW1[2 * _DN + _DE:]
    P, Q = pl.pallas_call(
        _pq_body,
        grid=(_N // _BN,),
        in_specs=[
            pl.BlockSpec((_BN, _DN), lambda i: (i, 0)),
            _full((_DN, _DH)),
            _full((_DN, _DH)),
        ],
        out_specs=[
            pl.BlockSpec((_BN, _DH), lambda i: (i, 0)),
            pl.BlockSpec((_BN, _DH), lambda i: (i, 0)),
        ],
        out_shape=[
            jax.ShapeDtypeStruct((_N, _DH), f32),
            jax.ShapeDtypeStruct((_N, _DH), f32),
        ],
    )(node_feat, Wvi, Wvj)

    G = jnp.take(P, src, axis=0) + jnp.take(Q, dst, axis=0)

    mij, uesum = pl.pallas_call(
        _edge_body,
        grid=(_E // _BE,),
        in_specs=[
            pl.BlockSpec((_BE, _DH), lambda i: (i, 0)),
            pl.BlockSpec((_BE, _DE), lambda i: (i, 0)),
            _full((1, _DS)),
            _full((_DS, _DH)),
            _full((_DE, _DH)),
            _full((1, _DH)),
            _full((_DH, _DH)),
            _full((1, _DH)),
        ],
        out_specs=[
            pl.BlockSpec((_BE, _DH), lambda i: (i, 0)),
            _full((1, _DH)),
        ],
        out_shape=[
            jax.ShapeDtypeStruct((_E, _DH), f32),
            jax.ShapeDtypeStruct((1, _DH), f32),
        ],
        scratch_shapes=[pltpu.VMEM((1, _DH), f32)],
    )(G, edge_feat, state_attr, Wu, We, eb1.reshape(1, _DH), eW2,
      eb2.reshape(1, _DH))

    esum = jax.ops.segment_sum(mij, dst, num_segments=_N)
    deg = jax.ops.segment_sum(jnp.ones((_E,), f32), dst, num_segments=_N)
    deg_b = jnp.broadcast_to(deg[:, None], (_N, _DH))

    v_new, snew = pl.pallas_call(
        _node_body,
        grid=(_N // _BN,),
        in_specs=[
            pl.BlockSpec((_BN, _DN), lambda i: (i, 0)),
            pl.BlockSpec((_BN, _DH), lambda i: (i, 0)),
            pl.BlockSpec((_BN, _DH), lambda i: (i, 0)),
            _full((1, _DS)),
            _full((_DN, _DH)),
            _full((_DH, _DH)),
            _full((_DS, _DH)),
            _full((1, _DH)),
            _full((_DH, _DH)),
            _full((1, _DH)),
            _full((1, _DH)),
            _full((_DS, _DH)),
            _full((_DH, _DH)),
            _full((_DH, _DH)),
            _full((1, _DH)),
            _full((_DH, _DS)),
            _full((1, _DS)),
        ],
        out_specs=[
            pl.BlockSpec((_BN, _DH), lambda i: (i, 0)),
            _full((1, _DS)),
        ],
        out_shape=[
            jax.ShapeDtypeStruct((_N, _DH), f32),
            jax.ShapeDtypeStruct((1, _DS), f32),
        ],
        scratch_shapes=[pltpu.VMEM((1, _DH), f32)],
    )(node_feat, esum, deg_b, state_attr,
      nW1[0:_DN], nW1[_DN:2 * _DN], nW1[2 * _DN:], nb1.reshape(1, _DH),
      nW2, nb2.reshape(1, _DH), uesum,
      sW1[0:_DS], sW1[_DS:_DS + _DH], sW1[_DS + _DH:], sb1.reshape(1, _DH),
      sW2, sb2.reshape(1, _DS))

    return (mij, v_new, snew.reshape(_DS))


# speedup vs baseline: 1.9646x; 1.7595x over previous
import functools

import jax
import jax.numpy as jnp
from jax import lax
from jax.experimental import pallas as pl
from jax.experimental.pallas import tpu as pltpu
from jax.experimental.pallas import tpu_sc as plsc

_N = 10000
_E = 320000
_DN = 128
_DE = 16
_DS = 64
_DH = 128

_BN = 2000
_BE = 3200

_NSC = 2
_NSUB = 16
_NW = _NSC * _NSUB
_EPW = _E // _NW
_GC = 80
_GCH = _EPW // _GC


def _gather_body(p_hbm, q_hbm, src_hbm, dst_hbm, g_hbm, si, di, pr, qr, sem):
    wid = lax.axis_index("s") * _NSC + lax.axis_index("c")
    base0 = wid * _EPW

    def chunk(k, carry):
        base = base0 + k * _GC
        pltpu.sync_copy(src_hbm.at[pl.ds(base, _GC)], si)
        pltpu.sync_copy(dst_hbm.at[pl.ds(base, _GC)], di)
        c1 = pltpu.async_copy(p_hbm.at[si], pr, sem)
        c2 = pltpu.async_copy(q_hbm.at[di], qr, sem)
        c1.wait()
        c2.wait()

        def addrow(r, cr):
            for cc in range(_DH // 16):
                s = pl.ds(cc * 16, 16)
                pr[r, s] = pr[r, s] + qr[r, s]
            return cr

        lax.fori_loop(0, _GC, addrow, 0)
        pltpu.sync_copy(pr, g_hbm.at[pl.ds(base, _GC)])
        return carry

    lax.fori_loop(0, _GCH, chunk, 0)


def _pq_body(nf, wa, wb, p_out, q_out):
    x = nf[...]
    p_out[...] = jnp.dot(x, wa[...], preferred_element_type=jnp.float32)
    q_out[...] = jnp.dot(x, wb[...], preferred_element_type=jnp.float32)


def _edge_body(g, ef, st, wu, we, eb1, ew2, eb2, mij, uesum, acc):
    i = pl.program_id(0)
    ce = jnp.dot(st[...], wu[...], preferred_element_type=jnp.float32) + eb1[...]
    h1 = jnp.maximum(
        g[...] + jnp.dot(ef[...], we[...], preferred_element_type=jnp.float32) + ce,
        0.0)
    m = jnp.maximum(
        jnp.dot(h1, ew2[...], preferred_element_type=jnp.float32) + eb2[...], 0.0)
    mij[...] = m

    @pl.when(i == 0)
    def _():
        acc[...] = jnp.zeros_like(acc)

    acc[...] += jnp.sum(m, axis=0, keepdims=True)

    @pl.when(i == pl.num_programs(0) - 1)
    def _():
        uesum[...] = acc[...]


def _node_body(nf, esum, deg, st, a_nf, a_ve, wun, nb1, nw2, nb2, uesum,
               sA, sB, sC, sb1, sw2, sb2, vnew, snew, accv):
    i = pl.program_id(0)
    ve = esum[...] / jnp.maximum(deg[...], 1.0)
    cn = jnp.dot(st[...], wun[...], preferred_element_type=jnp.float32) + nb1[...]
    h = jnp.maximum(
        jnp.dot(nf[...], a_nf[...], preferred_element_type=jnp.float32)
        + jnp.dot(ve, a_ve[...], preferred_element_type=jnp.float32) + cn, 0.0)
    v = jnp.maximum(
        jnp.dot(h, nw2[...], preferred_element_type=jnp.float32) + nb2[...], 0.0)
    vnew[...] = v

    @pl.when(i == 0)
    def _():
        accv[...] = jnp.zeros_like(accv)

    accv[...] += jnp.sum(v, axis=0, keepdims=True)

    @pl.when(i == pl.num_programs(0) - 1)
    def _():
        u_edge = uesum[...] * (1.0 / _E)
        u_vertex = accv[...] * (1.0 / _N)
        s1 = jnp.maximum(
            jnp.dot(st[...], sA[...], preferred_element_type=jnp.float32)
            + jnp.dot(u_edge, sB[...], preferred_element_type=jnp.float32)
            + jnp.dot(u_vertex, sC[...], preferred_element_type=jnp.float32)
            + sb1[...], 0.0)
        snew[...] = jnp.maximum(
            jnp.dot(s1, sw2[...], preferred_element_type=jnp.float32) + sb2[...],
            0.0)


def _full(shape):
    return pl.BlockSpec(shape, lambda *_: tuple(0 for _ in shape))


def kernel(node_feat, edge_feat, state_attr,
           eW1, eb1, eW2, eb2,
           nW1, nb1, nW2, nb2,
           sW1, sb1, sW2, sb2,
           edge_index):
    src = edge_index[0]
    dst = edge_index[1]
    f32 = jnp.float32

    Wvi = eW1[0:_DN]
    Wvj = eW1[_DN:2 * _DN]
    We = eW1[2 * _DN:2 * _DN + _DE]
    Wu = eW1[2 * _DN + _DE:]
    P, Q = pl.pallas_call(
        _pq_body,
        grid=(_N // _BN,),
        in_specs=[
            pl.BlockSpec((_BN, _DN), lambda i: (i, 0)),
            _full((_DN, _DH)),
            _full((_DN, _DH)),
        ],
        out_specs=[
            pl.BlockSpec((_BN, _DH), lambda i: (i, 0)),
            pl.BlockSpec((_BN, _DH), lambda i: (i, 0)),
        ],
        out_shape=[
            jax.ShapeDtypeStruct((_N, _DH), f32),
            jax.ShapeDtypeStruct((_N, _DH), f32),
        ],
    )(node_feat, Wvi, Wvj)

    G = pl.kernel(
        _gather_body,
        out_type=jax.ShapeDtypeStruct((_E, _DH), f32),
        mesh=plsc.VectorSubcoreMesh(core_axis_name="c", subcore_axis_name="s"),
        scratch_types=[
            pltpu.VMEM((_GC,), jnp.int32),
            pltpu.VMEM((_GC,), jnp.int32),
            pltpu.VMEM((_GC, _DH), f32),
            pltpu.VMEM((_GC, _DH), f32),
            pltpu.SemaphoreType.DMA,
        ],
    )(P, Q, src, dst)

    mij, uesum = pl.pallas_call(
        _edge_body,
        grid=(_E // _BE,),
        in_specs=[
            pl.BlockSpec((_BE, _DH), lambda i: (i, 0)),
            pl.BlockSpec((_BE, _DE), lambda i: (i, 0)),
            _full((1, _DS)),
            _full((_DS, _DH)),
            _full((_DE, _DH)),
            _full((1, _DH)),
            _full((_DH, _DH)),
            _full((1, _DH)),
        ],
        out_specs=[
            pl.BlockSpec((_BE, _DH), lambda i: (i, 0)),
            _full((1, _DH)),
        ],
        out_shape=[
            jax.ShapeDtypeStruct((_E, _DH), f32),
            jax.ShapeDtypeStruct((1, _DH), f32),
        ],
        scratch_shapes=[pltpu.VMEM((1, _DH), f32)],
    )(G, edge_feat, state_attr, Wu, We, eb1.reshape(1, _DH), eW2,
      eb2.reshape(1, _DH))

    esum = jax.ops.segment_sum(mij, dst, num_segments=_N)
    deg = jax.ops.segment_sum(jnp.ones((_E,), f32), dst, num_segments=_N)
    deg_b = jnp.broadcast_to(deg[:, None], (_N, _DH))

    v_new, snew = pl.pallas_call(
        _node_body,
        grid=(_N // _BN,),
        in_specs=[
            pl.BlockSpec((_BN, _DN), lambda i: (i, 0)),
            pl.BlockSpec((_BN, _DH), lambda i: (i, 0)),
            pl.BlockSpec((_BN, _DH), lambda i: (i, 0)),
            _full((1, _DS)),
            _full((_DN, _DH)),
            _full((_DH, _DH)),
            _full((_DS, _DH)),
            _full((1, _DH)),
            _full((_DH, _DH)),
            _full((1, _DH)),
            _full((1, _DH)),
            _full((_DS, _DH)),
            _full((_DH, _DH)),
            _full((_DH, _DH)),
            _full((1, _DH)),
            _full((_DH, _DS)),
            _full((1, _DS)),
        ],
        out_specs=[
            pl.BlockSpec((_BN, _DH), lambda i: (i, 0)),
            _full((1, _DS)),
        ],
        out_shape=[
            jax.ShapeDtypeStruct((_N, _DH), f32),
            jax.ShapeDtypeStruct((1, _DS), f32),
        ],
        scratch_shapes=[pltpu.VMEM((1, _DH), f32)],
    )(node_feat, esum, deg_b, state_attr,
      nW1[0:_DN], nW1[_DN:2 * _DN], nW1[2 * _DN:], nb1.reshape(1, _DH),
      nW2, nb2.reshape(1, _DH), uesum,
      sW1[0:_DS], sW1[_DS:_DS + _DH], sW1[_DS + _DH:], sb1.reshape(1, _DH),
      sW2, sb2.reshape(1, _DS))

    return (mij, v_new, snew.reshape(_DS))


# speedup vs baseline: 2.8159x; 1.4333x over previous
import functools

import jax
import jax.numpy as jnp
from jax import lax
from jax.experimental import pallas as pl
from jax.experimental.pallas import tpu as pltpu
from jax.experimental.pallas import tpu_sc as plsc

_N = 10000
_E = 320000
_DN = 128
_DE = 16
_DS = 64
_DH = 128

_BN = 2000
_BE = 3200

_NSC = 2
_NSUB = 16
_NW = _NSC * _NSUB
_EPW = _E // _NW
_GC = 80
_GCH = _EPW // _GC


def _gather_body(p_hbm, q_hbm, src_hbm, dst_hbm, g_hbm, si, di, pr, qr, sem):
    wid = lax.axis_index("s") * _NSC + lax.axis_index("c")
    base0 = wid * _EPW

    def chunk(k, carry):
        base = base0 + k * _GC
        pltpu.sync_copy(src_hbm.at[pl.ds(base, _GC)], si)
        pltpu.sync_copy(dst_hbm.at[pl.ds(base, _GC)], di)
        c1 = pltpu.async_copy(p_hbm.at[si], pr, sem)
        c2 = pltpu.async_copy(q_hbm.at[di], qr, sem)
        c1.wait()
        c2.wait()

        def addrow(r, cr):
            for cc in range(_DH // 16):
                s = pl.ds(cc * 16, 16)
                pr[r, s] = pr[r, s] + qr[r, s]
            return cr

        lax.fori_loop(0, _GC, addrow, 0)
        pltpu.sync_copy(pr, g_hbm.at[pl.ds(base, _GC)])
        return carry

    lax.fori_loop(0, _GCH, chunk, 0)


_NP = 10240
_NHALF = _NP // _NSC
_NTAB = _NHALF + 8
_NPT = _NHALF // _NSUB
_EPT = _E // _NSUB
_SCH = _EPT // _GC


def _remap(di, lo):
    lov = jnp.full((16,), 1, jnp.int32) * lo
    dump = jnp.full((16,), _NHALF, jnp.int32)
    for v in range(_GC // 16):
        s = pl.ds(v * 16, 16)
        rel = di[s] - lov
        ok = (rel >= 0) & (rel < _NHALF)
        di[s] = jnp.where(ok, rel, dump)


def _esum_body(mij_hbm, dst_hbm, esum_out, di, rows, zb, esum_s):
    cid = lax.axis_index("c")
    sid = lax.axis_index("s")
    lo = cid * _NHALF
    r0 = pl.multiple_of(sid * _NPT, 8)

    def fill_z(j, c):
        for cc in range(_DH // 16):
            zb[j, pl.ds(cc * 16, 16)] = jnp.zeros((16,), jnp.float32)
        return c

    lax.fori_loop(0, _NPT, fill_z, 0)
    pltpu.sync_copy(zb, esum_s.at[pl.ds(r0, _NPT)])
    plsc.subcore_barrier()

    def chunk(k, carry):
        base = sid * _EPT + k * _GC
        pltpu.sync_copy(dst_hbm.at[pl.ds(base, _GC)], di)
        pltpu.sync_copy(mij_hbm.at[pl.ds(base, _GC)], rows)
        _remap(di, lo)
        pltpu.sync_copy(rows, esum_s.at[di], add=True)
        return carry

    lax.fori_loop(0, _SCH, chunk, 0)
    plsc.subcore_barrier()

    out0 = pl.multiple_of(lo + sid * _NPT, 8)
    pltpu.sync_copy(esum_s.at[pl.ds(r0, _NPT)], esum_out.at[pl.ds(out0, _NPT)])


def _deg_body(dst_hbm, deg_out, di, ones_v, zb16, deg_s):
    cid = lax.axis_index("c")
    sid = lax.axis_index("s")
    lo = cid * _NHALF
    r0 = pl.multiple_of(sid * _NPT, 8)

    def fill(j, c):
        ones_v[j] = jnp.full((16,), 1.0, jnp.float32)
        return c

    lax.fori_loop(0, _GC, fill, 0)

    def fill16(j, c):
        zb16[j] = jnp.zeros((16,), jnp.float32)
        return c

    lax.fori_loop(0, _NPT, fill16, 0)
    pltpu.sync_copy(zb16, deg_s.at[pl.ds(r0, _NPT)])
    plsc.subcore_barrier()

    def chunk(k, carry):
        base = sid * _EPT + k * _GC
        pltpu.sync_copy(dst_hbm.at[pl.ds(base, _GC)], di)
        _remap(di, lo)
        pltpu.sync_copy(ones_v, deg_s.at[di], add=True)
        return carry

    lax.fori_loop(0, _SCH, chunk, 0)
    plsc.subcore_barrier()

    out0 = pl.multiple_of(lo + sid * _NPT, 8)
    pltpu.sync_copy(deg_s.at[pl.ds(r0, _NPT)], deg_out.at[pl.ds(out0, _NPT)])


def _pq_body(nf, wa, wb, p_out, q_out):
    x = nf[...]
    p_out[...] = jnp.dot(x, wa[...], preferred_element_type=jnp.float32)
    q_out[...] = jnp.dot(x, wb[...], preferred_element_type=jnp.float32)


def _edge_body(g, ef, st, wu, we, eb1, ew2, eb2, mij, uesum, acc):
    i = pl.program_id(0)
    ce = jnp.dot(st[...], wu[...], preferred_element_type=jnp.float32) + eb1[...]
    h1 = jnp.maximum(
        g[...] + jnp.dot(ef[...], we[...], preferred_element_type=jnp.float32) + ce,
        0.0)
    m = jnp.maximum(
        jnp.dot(h1, ew2[...], preferred_element_type=jnp.float32) + eb2[...], 0.0)
    mij[...] = m

    @pl.when(i == 0)
    def _():
        acc[...] = jnp.zeros_like(acc)

    acc[...] += jnp.sum(m, axis=0, keepdims=True)

    @pl.when(i == pl.num_programs(0) - 1)
    def _():
        uesum[...] = acc[...]


def _node_body(nf, esum, deg, st, a_nf, a_ve, wun, nb1, nw2, nb2, uesum,
               sA, sB, sC, sb1, sw2, sb2, vnew, snew, accv):
    i = pl.program_id(0)
    es = esum[...]
    dg = deg[...][:, 0:1]
    ve = es / jnp.maximum(dg, 1.0)
    cn = jnp.dot(st[...], wun[...], preferred_element_type=jnp.float32) + nb1[...]
    h = jnp.maximum(
        jnp.dot(nf[...], a_nf[...], preferred_element_type=jnp.float32)
        + jnp.dot(ve, a_ve[...], preferred_element_type=jnp.float32) + cn, 0.0)
    v = jnp.maximum(
        jnp.dot(h, nw2[...], preferred_element_type=jnp.float32) + nb2[...], 0.0)
    vnew[...] = v

    @pl.when(i == 0)
    def _():
        accv[...] = jnp.zeros_like(accv)

    accv[...] += jnp.sum(v, axis=0, keepdims=True)

    @pl.when(i == pl.num_programs(0) - 1)
    def _():
        u_edge = uesum[...] * (1.0 / _E)
        u_vertex = accv[...] * (1.0 / _N)
        s1 = jnp.maximum(
            jnp.dot(st[...], sA[...], preferred_element_type=jnp.float32)
            + jnp.dot(u_edge, sB[...], preferred_element_type=jnp.float32)
            + jnp.dot(u_vertex, sC[...], preferred_element_type=jnp.float32)
            + sb1[...], 0.0)
        snew[...] = jnp.maximum(
            jnp.dot(s1, sw2[...], preferred_element_type=jnp.float32) + sb2[...],
            0.0)


def _full(shape):
    return pl.BlockSpec(shape, lambda *_: tuple(0 for _ in shape))


def kernel(node_feat, edge_feat, state_attr,
           eW1, eb1, eW2, eb2,
           nW1, nb1, nW2, nb2,
           sW1, sb1, sW2, sb2,
           edge_index):
    src = edge_index[0]
    dst = edge_index[1]
    f32 = jnp.float32

    Wvi = eW1[0:_DN]
    Wvj = eW1[_DN:2 * _DN]
    We = eW1[2 * _DN:2 * _DN + _DE]
    Wu = eW1[2 * _DN + _DE:]
    P, Q = pl.pallas_call(
        _pq_body,
        grid=(_N // _BN,),
        in_specs=[
            pl.BlockSpec((_BN, _DN), lambda i: (i, 0)),
            _full((_DN, _DH)),
            _full((_DN, _DH)),
        ],
        out_specs=[
            pl.BlockSpec((_BN, _DH), lambda i: (i, 0)),
            pl.BlockSpec((_BN, _DH), lambda i: (i, 0)),
        ],
        out_shape=[
            jax.ShapeDtypeStruct((_N, _DH), f32),
            jax.ShapeDtypeStruct((_N, _DH), f32),
        ],
    )(node_feat, Wvi, Wvj)

    G = pl.kernel(
        _gather_body,
        out_type=jax.ShapeDtypeStruct((_E, _DH), f32),
        mesh=plsc.VectorSubcoreMesh(core_axis_name="c", subcore_axis_name="s"),
        scratch_types=[
            pltpu.VMEM((_GC,), jnp.int32),
            pltpu.VMEM((_GC,), jnp.int32),
            pltpu.VMEM((_GC, _DH), f32),
            pltpu.VMEM((_GC, _DH), f32),
            pltpu.SemaphoreType.DMA,
        ],
    )(P, Q, src, dst)

    mij, uesum = pl.pallas_call(
        _edge_body,
        grid=(_E // _BE,),
        in_specs=[
            pl.BlockSpec((_BE, _DH), lambda i: (i, 0)),
            pl.BlockSpec((_BE, _DE), lambda i: (i, 0)),
            _full((1, _DS)),
            _full((_DS, _DH)),
            _full((_DE, _DH)),
            _full((1, _DH)),
            _full((_DH, _DH)),
            _full((1, _DH)),
        ],
        out_specs=[
            pl.BlockSpec((_BE, _DH), lambda i: (i, 0)),
            _full((1, _DH)),
        ],
        out_shape=[
            jax.ShapeDtypeStruct((_E, _DH), f32),
            jax.ShapeDtypeStruct((1, _DH), f32),
        ],
        scratch_shapes=[pltpu.VMEM((1, _DH), f32)],
    )(G, edge_feat, state_attr, Wu, We, eb1.reshape(1, _DH), eW2,
      eb2.reshape(1, _DH))

    esum = pl.kernel(
        _esum_body,
        out_type=jax.ShapeDtypeStruct((_NP, _DH), f32),
        mesh=plsc.VectorSubcoreMesh(core_axis_name="c", subcore_axis_name="s"),
        scratch_types=[
            pltpu.VMEM((_GC,), jnp.int32),
            pltpu.VMEM((_GC, _DH), f32),
            pltpu.VMEM((_NPT, _DH), f32),
            pltpu.VMEM_SHARED((_NTAB, _DH), f32),
        ],
    )(mij, dst)
    deg = pl.kernel(
        _deg_body,
        out_type=jax.ShapeDtypeStruct((_NP, 16), f32),
        mesh=plsc.VectorSubcoreMesh(core_axis_name="c", subcore_axis_name="s"),
        scratch_types=[
            pltpu.VMEM((_GC,), jnp.int32),
            pltpu.VMEM((_GC, 16), f32),
            pltpu.VMEM((_NPT, 16), f32),
            pltpu.VMEM_SHARED((_NTAB, 16), f32),
        ],
    )(dst)

    v_new, snew = pl.pallas_call(
        _node_body,
        grid=(_N // _BN,),
        in_specs=[
            pl.BlockSpec((_BN, _DN), lambda i: (i, 0)),
            pl.BlockSpec((_BN, _DH), lambda i: (i, 0)),
            pl.BlockSpec((_BN, 16), lambda i: (i, 0)),
            _full((1, _DS)),
            _full((_DN, _DH)),
            _full((_DH, _DH)),
            _full((_DS, _DH)),
            _full((1, _DH)),
            _full((_DH, _DH)),
            _full((1, _DH)),
            _full((1, _DH)),
            _full((_DS, _DH)),
            _full((_DH, _DH)),
            _full((_DH, _DH)),
            _full((1, _DH)),
            _full((_DH, _DS)),
            _full((1, _DS)),
        ],
        out_specs=[
            pl.BlockSpec((_BN, _DH), lambda i: (i, 0)),
            _full((1, _DS)),
        ],
        out_shape=[
            jax.ShapeDtypeStruct((_N, _DH), f32),
            jax.ShapeDtypeStruct((1, _DS), f32),
        ],
        scratch_shapes=[pltpu.VMEM((1, _DH), f32)],
    )(node_feat, esum, deg, state_attr,
      nW1[0:_DN], nW1[_DN:2 * _DN], nW1[2 * _DN:], nb1.reshape(1, _DH),
      nW2, nb2.reshape(1, _DH), uesum,
      sW1[0:_DS], sW1[_DS:_DS + _DH], sW1[_DS + _DH:], sb1.reshape(1, _DH),
      sW2, sb2.reshape(1, _DS))

    return (mij, v_new, snew.reshape(_DS))
